# Initial kernel scaffold; baseline (speedup 1.0000x reference)
import jax
import jax.numpy as jnp
from jax.experimental import pallas as pl


def kernel(x, emb, Wb0, bb0, Wb1, bb1, Wb2, bb2, Wt0, bt0, Wt1, bt1, Wt2, bt2, Wt3, bt3, Wt4, bt4):
    raise NotImplementedError("write your pallas kernel here")


# speedup vs baseline: 8.1839x; 8.1839x over previous
import functools

import jax
import jax.numpy as jnp
import numpy as np
from jax import lax
from jax.experimental import pallas as pl
from jax.experimental.pallas import tpu as pltpu
from jax.experimental.pallas import tpu_sc as plsc

B = 4096
ND = 13
NS = 26
V = 100000
E = 128
NF = NS + 1
P = 32
H0 = 1024

NW = 32
ROWS = B * NS
RPW = ROWS // NW
CHUNK = 128
NCHUNK = RPW // CHUNK

@functools.cache
def _make_sc_gather():
    mesh = plsc.VectorSubcoreMesh(core_axis_name="c", subcore_axis_name="s")

    @functools.partial(
        pl.kernel,
        mesh=mesh,
        out_type=jax.ShapeDtypeStruct((ROWS, E), jnp.float32),
        scratch_types=[
            pltpu.VMEM((NCHUNK, CHUNK), jnp.int32),
            pltpu.VMEM((CHUNK, E), jnp.float32),
            pltpu.VMEM((CHUNK, E), jnp.float32),
            pltpu.SemaphoreType.DMA,
            pltpu.SemaphoreType.DMA,
        ],
    )
    def _sc_gather(emb_hbm, idx_hbm, out_hbm, idx_v, rows_a, rows_b, sem_a, sem_b):
        wid = lax.axis_index("s") * 2 + lax.axis_index("c")
        base = wid * RPW
        pltpu.sync_copy(idx_hbm.at[wid], idx_v)

        def body(p, carry):
            c0 = 2 * p
            ha = pltpu.async_copy(emb_hbm.at[idx_v.at[c0]], rows_a, sem_a)
            hb = pltpu.async_copy(emb_hbm.at[idx_v.at[c0 + 1]], rows_b, sem_b)
            ha.wait()
            pltpu.sync_copy(rows_a, out_hbm.at[pl.ds(base + c0 * CHUNK, CHUNK)])
            hb.wait()
            pltpu.sync_copy(rows_b, out_hbm.at[pl.ds(base + (c0 + 1) * CHUNK, CHUNK)])
            return carry

        lax.fori_loop(0, NCHUNK // 2, body, 0)

    return _sc_gather


BB = 256
_IU0, _IU1 = np.triu_indices(NF)
_PAIR_SCALE = np.float32(0.5)


def _tc_body(x_ref, eb_ref, wb0, bb0, wb1, bb1, wb2, bb2,
             w0a, w0g, bt0, wt1, bt1, wt2, bt2, wt3, bt3, wt4, bt4, o_ref):
    f32 = jnp.float32
    dense = x_ref[:, :ND]
    h = jnp.maximum(jnp.dot(dense, wb0[:], preferred_element_type=f32) + bb0[:], 0.0)
    h = jnp.maximum(jnp.dot(h, wb1[:], preferred_element_type=f32) + bb1[:], 0.0)
    bot = jnp.maximum(jnp.dot(h, wb2[:], preferred_element_type=f32) + bb2[:], 0.0)
    feats = jnp.concatenate(
        [bot.reshape(BB, 1, E), eb_ref[:], jnp.zeros((BB, P - NF, E), f32)], axis=1)
    gram = lax.dot_general(feats, feats, (((2,), (2,)), ((0,), (0,))),
                           preferred_element_type=f32)
    gflat = gram.reshape(BB, P * P)
    h = jnp.dot(bot, w0a[:], preferred_element_type=f32)
    h = h + jnp.dot(gflat, w0g[:], preferred_element_type=f32)
    h = jnp.maximum(h + bt0[:], 0.0)
    h = jnp.maximum(jnp.dot(h, wt1[:], preferred_element_type=f32) + bt1[:], 0.0)
    h = jnp.maximum(jnp.dot(h, wt2[:], preferred_element_type=f32) + bt2[:], 0.0)
    h = jnp.maximum(jnp.dot(h, wt3[:], preferred_element_type=f32) + bt3[:], 0.0)
    o_ref[:, :] = jnp.dot(h, wt4[:], preferred_element_type=f32) + bt4[:]


def _const_spec(shape):
    nd = len(shape)
    return pl.BlockSpec(shape, lambda i: (0,) * nd)


def _tc_forward(x, embed, wb0, bb0, wb1, bb1, wb2, bb2,
                w0a, w0g, bt0, wt1, bt1, wt2, bt2, wt3, bt3, wt4, bt4):
    nblk = B // BB
    consts = [wb0, bb0, wb1, bb1, wb2, bb2, w0a, w0g, bt0,
              wt1, bt1, wt2, bt2, wt3, bt3, wt4, bt4]
    in_specs = [
        pl.BlockSpec((BB, ND + NS), lambda i: (i, 0)),
        pl.BlockSpec((BB, NS, E), lambda i: (i, 0, 0)),
    ] + [_const_spec(c.shape) for c in consts]
    return pl.pallas_call(
        _tc_body,
        grid=(nblk,),
        in_specs=in_specs,
        out_specs=pl.BlockSpec((BB, 1), lambda i: (i, 0)),
        out_shape=jax.ShapeDtypeStruct((B, 1), jnp.float32),
        compiler_params=pltpu.CompilerParams(
            dimension_semantics=("arbitrary",)),
    )(x, embed, *consts)


def kernel(x, emb, Wb0, bb0, Wb1, bb1, Wb2, bb2,
           Wt0, bt0, Wt1, bt1, Wt2, bt2, Wt3, bt3, Wt4, bt4):
    cat = x[:, ND:].astype(jnp.int32)
    idx = (cat + (jnp.arange(NS, dtype=jnp.int32) * V)[None, :]).reshape(
        NW, NCHUNK, CHUNK)

    wp = Wt0[E:] * _PAIR_SCALE
    w0g = jnp.zeros((P, P, H0), jnp.float32)
    w0g = w0g.at[_IU0, _IU1].add(wp).at[_IU1, _IU0].add(wp)
    w0g = w0g.reshape(P * P, H0)
    w0a = Wt0[:E]

    embed = _make_sc_gather()(emb, idx).reshape(B, NS, E)

    out = _tc_forward(
        x, embed, Wb0, bb0.reshape(1, -1), Wb1, bb1.reshape(1, -1),
        Wb2, bb2.reshape(1, -1), w0a, w0g, bt0.reshape(1, -1),
        Wt1, bt1.reshape(1, -1), Wt2, bt2.reshape(1, -1),
        Wt3, bt3.reshape(1, -1), Wt4, bt4.reshape(1, -1))
    return out


# speedup vs baseline: 10.1126x; 1.2357x over previous
import functools

import jax
import jax.numpy as jnp
import numpy as np
from jax import lax
from jax.experimental import pallas as pl
from jax.experimental.pallas import tpu as pltpu
from jax.experimental.pallas import tpu_sc as plsc

B = 4096
ND = 13
NS = 26
V = 100000
E = 128
NF = NS + 1
P = 32
H0 = 1024

NW = 32
ROWS = B * NS
RPW = ROWS // NW
CHUNK = 128
NCHUNK = RPW // CHUNK

@functools.cache
def _make_sc_gather():
    mesh = plsc.VectorSubcoreMesh(core_axis_name="c", subcore_axis_name="s")

    @functools.partial(
        pl.kernel,
        mesh=mesh,
        out_type=jax.ShapeDtypeStruct((ROWS, E), jnp.float32),
        scratch_types=[
            pltpu.VMEM((NCHUNK, CHUNK), jnp.int32),
            pltpu.VMEM((CHUNK, E), jnp.float32),
            pltpu.VMEM((CHUNK, E), jnp.float32),
            pltpu.SemaphoreType.DMA,
            pltpu.SemaphoreType.DMA,
        ],
    )
    def _sc_gather(emb_hbm, idx_hbm, out_hbm, idx_v, rows_a, rows_b, sem_a, sem_b):
        wid = lax.axis_index("s") * 2 + lax.axis_index("c")
        base = wid * RPW
        pltpu.sync_copy(idx_hbm.at[wid], idx_v)

        def body(p, carry):
            c0 = 2 * p
            ha = pltpu.async_copy(emb_hbm.at[idx_v.at[c0]], rows_a, sem_a)
            hb = pltpu.async_copy(emb_hbm.at[idx_v.at[c0 + 1]], rows_b, sem_b)
            ha.wait()
            pltpu.sync_copy(rows_a, out_hbm.at[pl.ds(base + c0 * CHUNK, CHUNK)])
            hb.wait()
            pltpu.sync_copy(rows_b, out_hbm.at[pl.ds(base + (c0 + 1) * CHUNK, CHUNK)])
            return carry

        lax.fori_loop(0, NCHUNK // 2, body, 0)

    return _sc_gather


BB = 256
_IU0, _IU1 = np.triu_indices(NF)
NPAIR = _IU0.shape[0]
NPAD = 384

_SEL_NP = np.zeros((P * P, NPAD), np.float32)
_SEL_NP[_IU0 * P + _IU1, np.arange(NPAIR)] += 0.5
_SEL_NP[_IU1 * P + _IU0, np.arange(NPAIR)] += 0.5


def _tc_body(x_ref, eb_ref, wb0, bb0, wb1, bb1, wb2, bb2,
             w0a, sel, w0p, bt0, wt1, bt1, wt2, bt2, wt3, bt3, wt4, bt4, o_ref):
    f32 = jnp.float32
    dense = x_ref[:, :ND]
    h = jnp.maximum(jnp.dot(dense, wb0[:], preferred_element_type=f32) + bb0[:], 0.0)
    h = jnp.maximum(jnp.dot(h, wb1[:], preferred_element_type=f32) + bb1[:], 0.0)
    bot = jnp.maximum(jnp.dot(h, wb2[:], preferred_element_type=f32) + bb2[:], 0.0)
    feats = jnp.concatenate(
        [bot.reshape(BB, 1, E), eb_ref[:], jnp.zeros((BB, P - NF, E), f32)], axis=1)
    gram = lax.dot_general(feats, feats, (((2,), (2,)), ((0,), (0,))),
                           preferred_element_type=f32)
    gflat = gram.reshape(BB, P * P)
    acts = jnp.dot(gflat, sel[:], preferred_element_type=f32)
    h = jnp.dot(bot, w0a[:], preferred_element_type=f32)
    h = h + jnp.dot(acts, w0p[:], preferred_element_type=f32)
    h = jnp.maximum(h + bt0[:], 0.0)
    h = jnp.maximum(jnp.dot(h, wt1[:], preferred_element_type=f32) + bt1[:], 0.0)
    h = jnp.maximum(jnp.dot(h, wt2[:], preferred_element_type=f32) + bt2[:], 0.0)
    h = jnp.maximum(jnp.dot(h, wt3[:], preferred_element_type=f32) + bt3[:], 0.0)
    o_ref[:, :] = jnp.dot(h, wt4[:], preferred_element_type=f32) + bt4[:]


def _const_spec(shape):
    nd = len(shape)
    return pl.BlockSpec(shape, lambda i: (0,) * nd)


def _tc_forward(x, embed, wb0, bb0, wb1, bb1, wb2, bb2,
                w0a, sel, w0p, bt0, wt1, bt1, wt2, bt2, wt3, bt3, wt4, bt4):
    nblk = B // BB
    consts = [wb0, bb0, wb1, bb1, wb2, bb2, w0a, sel, w0p, bt0,
              wt1, bt1, wt2, bt2, wt3, bt3, wt4, bt4]
    in_specs = [
        pl.BlockSpec((BB, ND + NS), lambda i: (i, 0)),
        pl.BlockSpec((BB, NS, E), lambda i: (i, 0, 0)),
    ] + [_const_spec(c.shape) for c in consts]
    return pl.pallas_call(
        _tc_body,
        grid=(nblk,),
        in_specs=in_specs,
        out_specs=pl.BlockSpec((BB, 1), lambda i: (i, 0)),
        out_shape=jax.ShapeDtypeStruct((B, 1), jnp.float32),
        compiler_params=pltpu.CompilerParams(
            dimension_semantics=("arbitrary",)),
    )(x, embed, *consts)


def kernel(x, emb, Wb0, bb0, Wb1, bb1, Wb2, bb2,
           Wt0, bt0, Wt1, bt1, Wt2, bt2, Wt3, bt3, Wt4, bt4):
    cat = x[:, ND:].astype(jnp.int32)
    idx = (cat + (jnp.arange(NS, dtype=jnp.int32) * V)[None, :]).reshape(
        NW, NCHUNK, CHUNK)

    sel = jnp.asarray(_SEL_NP)
    w0p = jnp.concatenate(
        [Wt0[E:], jnp.zeros((NPAD - NPAIR, H0), jnp.float32)], axis=0)
    w0a = Wt0[:E]

    embed = _make_sc_gather()(emb, idx).reshape(B, NS, E)

    out = _tc_forward(
        x, embed, Wb0, bb0.reshape(1, -1), Wb1, bb1.reshape(1, -1),
        Wb2, bb2.reshape(1, -1), w0a, sel, w0p, bt0.reshape(1, -1),
        Wt1, bt1.reshape(1, -1), Wt2, bt2.reshape(1, -1),
        Wt3, bt3.reshape(1, -1), Wt4, bt4.reshape(1, -1))
    return out
